# Initial kernel scaffold; baseline (speedup 1.0000x reference)
import jax
import jax.numpy as jnp
from jax.experimental import pallas as pl


def kernel(x, table):
    raise NotImplementedError("write your pallas kernel here")


# speedup vs baseline: 2.8556x; 2.8556x over previous
import functools
import math

import jax
import jax.numpy as jnp
from jax import lax
from jax.experimental import pallas as pl
from jax.experimental.pallas import tpu as pltpu
from jax.experimental.pallas import tpu_sc as plsc

_NC = 2
_NS = 16
_LANES = 16
_NW = _NC * _NS

_CH = 128


@functools.partial(jax.jit, static_argnames=("b_total", "d_model"))
def _emb_lookup(x_flat, table, *, b_total, d_model):
    b_per_w = b_total // _NW
    n_chunks = b_per_w // _CH
    scale = jnp.float32(math.sqrt(float(d_model)))
    vecs_per_row = d_model // _LANES

    mesh = plsc.VectorSubcoreMesh(core_axis_name="c", subcore_axis_name="s")

    @functools.partial(
        pl.kernel,
        mesh=mesh,
        out_type=jax.ShapeDtypeStruct((b_total, d_model), jnp.float32),
        scratch_types=[
            pltpu.VMEM((b_per_w,), jnp.int32),
            pltpu.VMEM((_CH, d_model), jnp.float32),
            pltpu.VMEM((_CH, d_model), jnp.float32),
            pltpu.SemaphoreType.DMA,
            pltpu.SemaphoreType.DMA,
            pltpu.SemaphoreType.DMA,
            pltpu.SemaphoreType.DMA,
        ],
    )
    def body(idx_hbm, table_hbm, out_hbm, idx_v, buf0, buf1, g0, g1, s0, s1):
        wid = lax.axis_index("s") * _NC + lax.axis_index("c")
        base = wid * b_per_w
        pltpu.sync_copy(idx_hbm.at[pl.ds(base, b_per_w)], idx_v)

        bufs = (buf0, buf1)
        gsems = (g0, g1)
        ssems = (s0, s1)

        def gather_desc(c):
            b = c % 2
            return pltpu.make_async_copy(
                table_hbm.at[idx_v.at[pl.ds(c * _CH, _CH)]], bufs[b], gsems[b]
            )

        def store_desc(c):
            b = c % 2
            return pltpu.make_async_copy(
                bufs[b], out_hbm.at[pl.ds(base + c * _CH, _CH)], ssems[b]
            )

        def scale_chunk(b):
            buf = bufs[b]

            def row(r, carry):
                for j in range(vecs_per_row):
                    sl = (r, pl.ds(j * _LANES, _LANES))
                    buf[sl] = buf[sl] * scale
                return carry

            lax.fori_loop(0, _CH, row, 0, unroll=2)

        gather_desc(0).start()
        for c in range(n_chunks):
            if c + 1 < n_chunks:
                if c >= 1:
                    store_desc(c - 1).wait()
                gather_desc(c + 1).start()
            gather_desc(c).wait()
            scale_chunk(c % 2)
            store_desc(c).start()
        store_desc(n_chunks - 1).wait()
        if n_chunks >= 2:
            store_desc(n_chunks - 2).wait()

    return body(x_flat, table)


def kernel(x, table):
    b_total = x.shape[0] * x.shape[1]
    d_model = table.shape[1]
    x_flat = x.reshape(b_total).astype(jnp.int32)
    out = _emb_lookup(x_flat, table, b_total=b_total, d_model=d_model)
    return out.reshape(x.shape[0], x.shape[1], d_model)
